# baseline (speedup 1.0000x reference)
import functools

import jax
import jax.numpy as jnp
from jax import lax
from jax.experimental import pallas as pl
from jax.experimental.pallas import tpu as pltpu
from jax.experimental.pallas import tpu_sc as plsc


def _sc_gather(emb_table, x):
    info = plsc.get_sparse_core_info()
    nc, ns = info.num_cores, info.num_subcores
    nw = nc * ns
    B = x.shape[0]
    D = emb_table.shape[1]
    assert B % (8 * nw) == 0
    b_per_w = B // nw
    mesh = plsc.VectorSubcoreMesh(core_axis_name="c", subcore_axis_name="s")

    @functools.partial(
        pl.kernel,
        mesh=mesh,
        out_type=jax.ShapeDtypeStruct((B, D), jnp.float32),
        scratch_types=[
            pltpu.VMEM((b_per_w,), jnp.int32),
            pltpu.VMEM((b_per_w, D), jnp.float32),
            pltpu.SemaphoreType.DMA,
        ],
    )
    def gather_kernel(table_hbm, idx_hbm, out_hbm, idx_v, rows_v, sem):
        wid = lax.axis_index("s") * nc + lax.axis_index("c")
        base = wid * b_per_w
        pltpu.sync_copy(idx_hbm.at[pl.ds(base, b_per_w)], idx_v)
        pltpu.async_copy(table_hbm.at[idx_v], rows_v, sem).wait()
        pltpu.sync_copy(rows_v, out_hbm.at[pl.ds(base, b_per_w)])

    return gather_kernel(emb_table, x)


_TN = 5120
_KB = 10


def _cast_kernel(w_ref, out_ref):
    out_ref[...] = w_ref[...].astype(jnp.bfloat16)


def _precast(W):
    D = W.shape[1]
    return pl.pallas_call(
        _cast_kernel,
        grid=(_KB,),
        in_specs=[pl.BlockSpec((_TN, D), lambda i: (i, 0))],
        out_specs=pl.BlockSpec((_TN, D), lambda i: (i, 0)),
        out_shape=jax.ShapeDtypeStruct((_KB * _TN, D), jnp.bfloat16),
        compiler_params=pltpu.CompilerParams(
            dimension_semantics=("arbitrary",),
        ),
    )(W)


def _proj_kernel(wb_ref, wf_ref, emb_ref, b_ref, out_ref):
    i = pl.program_id(0)
    emb = emb_ref[...].astype(jnp.bfloat16)
    bcol = b_ref[...][:, None]

    @pl.when(i < _KB)
    def _():
        acc = lax.dot_general(
            wb_ref[...], emb, (((1,), (1,)), ((), ())),
            preferred_element_type=jnp.float32,
        )
        out_ref[...] = acc + bcol

    @pl.when(i >= _KB)
    def _():
        acc = lax.dot_general(
            wf_ref[...].astype(jnp.bfloat16), emb, (((1,), (1,)), ((), ())),
            preferred_element_type=jnp.float32,
        )
        out_ref[...] = acc + bcol


def _tc_project(emb, W_bf16, W, b):
    B, D = emb.shape
    V = W.shape[0]
    nb = pl.cdiv(V, _TN)
    out_t = pl.pallas_call(
        _proj_kernel,
        grid=(nb,),
        in_specs=[
            pl.BlockSpec((_TN, D), lambda i: (jnp.minimum(i, _KB - 1), 0)),
            pl.BlockSpec((_TN, D), lambda i: (jnp.maximum(i, _KB), 0)),
            pl.BlockSpec((B, D), lambda i: (0, 0)),
            pl.BlockSpec((_TN,), lambda i: (i,)),
        ],
        out_specs=pl.BlockSpec((_TN, B), lambda i: (i, 0)),
        out_shape=jax.ShapeDtypeStruct((V, B), jnp.float32),
        compiler_params=pltpu.CompilerParams(
            dimension_semantics=("arbitrary",),
        ),
    )(W_bf16, W, emb, b)
    return out_t.T


def kernel(x, emb_table, W, b):
    emb = _sc_gather(emb_table, x)
    W_bf16 = _precast(W)
    return _tc_project(emb, W_bf16, W, b)


# speedup vs baseline: 1.0441x; 1.0441x over previous
import functools

import jax
import jax.numpy as jnp
from jax import lax
from jax.experimental import pallas as pl
from jax.experimental.pallas import tpu as pltpu
from jax.experimental.pallas import tpu_sc as plsc


def _sc_gather(emb_table, x):
    info = plsc.get_sparse_core_info()
    nc, ns = info.num_cores, info.num_subcores
    nw = nc * ns
    B = x.shape[0]
    D = emb_table.shape[1]
    assert B % (8 * nw) == 0
    b_per_w = B // nw
    mesh = plsc.VectorSubcoreMesh(core_axis_name="c", subcore_axis_name="s")

    @functools.partial(
        pl.kernel,
        mesh=mesh,
        out_type=jax.ShapeDtypeStruct((B, D), jnp.float32),
        scratch_types=[
            pltpu.VMEM((b_per_w,), jnp.int32),
            pltpu.VMEM((b_per_w, D), jnp.float32),
            pltpu.SemaphoreType.DMA,
        ],
    )
    def gather_kernel(table_hbm, idx_hbm, out_hbm, idx_v, rows_v, sem):
        wid = lax.axis_index("s") * nc + lax.axis_index("c")
        base = wid * b_per_w
        pltpu.sync_copy(idx_hbm.at[pl.ds(base, b_per_w)], idx_v)
        pltpu.async_copy(table_hbm.at[idx_v], rows_v, sem).wait()
        pltpu.sync_copy(rows_v, out_hbm.at[pl.ds(base, b_per_w)])

    return gather_kernel(emb_table, x)


_TN = 5120


def _proj_kernel(w_ref, emb_ref, b_ref, out_ref):
    emb = emb_ref[...].astype(jnp.bfloat16)
    w = w_ref[...].astype(jnp.bfloat16)
    acc = lax.dot_general(
        w, emb, (((1,), (1,)), ((), ())), preferred_element_type=jnp.float32
    )
    out_ref[...] = acc + b_ref[...][:, None]


def _tc_project(emb, W, b):
    B, D = emb.shape
    V = W.shape[0]
    nb = pl.cdiv(V, _TN)
    out_t = pl.pallas_call(
        _proj_kernel,
        grid=(nb,),
        in_specs=[
            pl.BlockSpec((_TN, D), lambda i: (i, 0)),
            pl.BlockSpec((B, D), lambda i: (0, 0)),
            pl.BlockSpec((_TN,), lambda i: (i,)),
        ],
        out_specs=pl.BlockSpec((_TN, B), lambda i: (i, 0)),
        out_shape=jax.ShapeDtypeStruct((V, B), jnp.float32),
        compiler_params=pltpu.CompilerParams(
            dimension_semantics=("arbitrary",),
        ),
    )(W, emb, b)
    return out_t.T


def kernel(x, emb_table, W, b):
    emb = _sc_gather(emb_table, x)
    return _tc_project(emb, W, b)
